# Initial kernel scaffold; baseline (speedup 1.0000x reference)
import jax
import jax.numpy as jnp
from jax.experimental import pallas as pl


def kernel(edge_attr, table):
    raise NotImplementedError("write your pallas kernel here")


# speedup vs baseline: 2.6095x; 2.6095x over previous
import functools

import jax
import jax.numpy as jnp
from jax import lax
from jax.experimental import pallas as pl
from jax.experimental.pallas import tpu as pltpu
from jax.experimental.pallas import tpu_sc as plsc

NUM_TYPES = 32
EMB_DIM = 16
N_EDGES = 3200000

_info = plsc.get_sparse_core_info()
_NC, _NS = _info.num_cores, _info.num_subcores
_NW = _NC * _NS
_PER_W = N_EDGES // _NW
_CHUNK = 2000
_NITER = _PER_W // _CHUNK


def _make_kernel():
    mesh = plsc.VectorSubcoreMesh(core_axis_name="c", subcore_axis_name="s")

    @functools.partial(
        pl.kernel,
        mesh=mesh,
        compiler_params=pltpu.CompilerParams(use_tc_tiling_on_sc=False),
        out_type=jax.ShapeDtypeStruct((N_EDGES, EMB_DIM), jnp.float32),
        scratch_types=[
            pltpu.VMEM((_CHUNK,), jnp.int32),
            pltpu.VMEM((_CHUNK, EMB_DIM), jnp.float32),
            pltpu.SemaphoreType.DMA,
        ],
    )
    def gather_kernel(table_hbm, idx_hbm, out_hbm, idx_v, rows_v, sem):
        wid = lax.axis_index("s") * _NC + lax.axis_index("c")
        w_base = wid * _PER_W

        def body(i, carry):
            base = w_base + i * _CHUNK
            pltpu.sync_copy(idx_hbm.at[pl.ds(base, _CHUNK)], idx_v)
            pltpu.async_copy(table_hbm.at[idx_v], rows_v, sem).wait()
            pltpu.sync_copy(rows_v, out_hbm.at[pl.ds(base, _CHUNK)])
            return carry

        lax.fori_loop(0, _NITER, body, 0)

    return gather_kernel


_gather = _make_kernel()


def kernel(edge_attr, table):
    return _gather(table, edge_attr)


# speedup vs baseline: 2.6159x; 1.0025x over previous
import functools

import jax
import jax.numpy as jnp
from jax import lax
from jax.experimental import pallas as pl
from jax.experimental.pallas import tpu as pltpu
from jax.experimental.pallas import tpu_sc as plsc

NUM_TYPES = 32
EMB_DIM = 16
N_EDGES = 3200000

_info = plsc.get_sparse_core_info()
_NC, _NS = _info.num_cores, _info.num_subcores
_NW = _NC * _NS
_PER_W = N_EDGES // _NW
_CHUNK = 2000
_NITER = _PER_W // _CHUNK
_NHALF = _NITER // 2


def _make_kernel():
    mesh = plsc.VectorSubcoreMesh(core_axis_name="c", subcore_axis_name="s")

    @functools.partial(
        pl.kernel,
        mesh=mesh,
        compiler_params=pltpu.CompilerParams(use_tc_tiling_on_sc=False),
        out_type=jax.ShapeDtypeStruct((N_EDGES, EMB_DIM), jnp.float32),
        scratch_types=[
            pltpu.VMEM((_CHUNK,), jnp.int32),
            pltpu.VMEM((_CHUNK,), jnp.int32),
            pltpu.VMEM((_CHUNK, EMB_DIM), jnp.float32),
            pltpu.VMEM((_CHUNK, EMB_DIM), jnp.float32),
        ] + [pltpu.SemaphoreType.DMA] * 6,
    )
    def gather_kernel(table_hbm, idx_hbm, out_hbm,
                      idx0, idx1, rows0, rows1, si0, si1, sg0, sg1, so0, so1):
        wid = lax.axis_index("s") * _NC + lax.axis_index("c")
        w_base = wid * _PER_W
        idx_b, rows_b = (idx0, idx1), (rows0, rows1)
        si, sg, so = (si0, si1), (sg0, sg1), (so0, so1)

        def base(g):
            return w_base + g * _CHUNK

        pltpu.async_copy(idx_hbm.at[pl.ds(base(0), _CHUNK)], idx0, si0)

        def step(g, b, not_first, not_last):
            pltpu.make_async_copy(
                idx_hbm.at[pl.ds(base(g), _CHUNK)], idx_b[b], si[b]).wait()

            def wait_prev_out():
                pltpu.make_async_copy(
                    rows_b[b], out_hbm.at[pl.ds(base(g - 2), _CHUNK)],
                    so[b]).wait()
            if not_first is None:
                wait_prev_out()
            else:
                pl.when(not_first)(wait_prev_out)

            gather = pltpu.async_copy(table_hbm.at[idx_b[b]], rows_b[b], sg[b])

            def next_idx_load():
                pltpu.async_copy(
                    idx_hbm.at[pl.ds(base(g + 1), _CHUNK)],
                    idx_b[1 - b], si[1 - b])
            if not_last is None:
                next_idx_load()
            else:
                pl.when(not_last)(next_idx_load)

            gather.wait()
            pltpu.async_copy(rows_b[b], out_hbm.at[pl.ds(base(g), _CHUNK)],
                             so[b])

        def body(i, carry):
            g = 2 * i
            step(g, 0, i >= 1, None)
            step(g + 1, 1, i >= 1, i < _NHALF - 1)
            return carry

        lax.fori_loop(0, _NHALF, body, 0)

        pltpu.make_async_copy(
            rows0, out_hbm.at[pl.ds(base(_NITER - 2), _CHUNK)], so0).wait()
        pltpu.make_async_copy(
            rows1, out_hbm.at[pl.ds(base(_NITER - 1), _CHUNK)], so1).wait()

    return gather_kernel


_gather = _make_kernel()


def kernel(edge_attr, table):
    return _gather(table, edge_attr)


# speedup vs baseline: 6.0535x; 2.3141x over previous
import functools

import jax
import jax.numpy as jnp
from jax import lax
from jax.experimental import pallas as pl
from jax.experimental.pallas import tpu as pltpu
from jax.experimental.pallas import tpu_sc as plsc

NUM_TYPES = 32
EMB_DIM = 16
N_EDGES = 3200000

_info = plsc.get_sparse_core_info()
_NC, _NS = _info.num_cores, _info.num_subcores
_NW = _NC * _NS
_PER_W = N_EDGES // _NW
_CHUNK = 2000
_NITER = _PER_W // _CHUNK
_NHALF = _NITER // 2
_GROUPS = _CHUNK // 16


def _make_kernel():
    mesh = plsc.VectorSubcoreMesh(core_axis_name="c", subcore_axis_name="s")

    @functools.partial(
        pl.kernel,
        mesh=mesh,
        compiler_params=pltpu.CompilerParams(
            use_tc_tiling_on_sc=False, needs_layout_passes=False),
        out_type=jax.ShapeDtypeStruct((N_EDGES, EMB_DIM), jnp.float32),
        scratch_types=[
            pltpu.VMEM((NUM_TYPES, EMB_DIM), jnp.float32),
            pltpu.VMEM((_CHUNK,), jnp.int32),
            pltpu.VMEM((_CHUNK,), jnp.int32),
            pltpu.VMEM((_CHUNK, EMB_DIM), jnp.float32),
            pltpu.VMEM((_CHUNK, EMB_DIM), jnp.float32),
        ] + [pltpu.SemaphoreType.DMA] * 4,
    )
    def gather_kernel(table_hbm, idx_hbm, out_hbm,
                      table_v, idx0, idx1, rows0, rows1, si0, si1, so0, so1):
        wid = lax.axis_index("s") * _NC + lax.axis_index("c")
        w_base = wid * _PER_W
        idx_b, rows_b = (idx0, idx1), (rows0, rows1)
        si, so = (si0, si1), (so0, so1)

        def base(g):
            return w_base + g * _CHUNK

        pltpu.sync_copy(table_hbm, table_v)
        pltpu.async_copy(idx_hbm.at[pl.ds(base(0), _CHUNK)], idx0, si0)

        lanes = lax.iota(jnp.int32, 16)

        def compute_chunk(idx_ref, rows_ref):
            def jbody(j, carry):
                idxvec = idx_ref[pl.ds(j * 16, 16)]
                rowvec = j * 16 + lanes
                for c in range(EMB_DIM):
                    csplat = jnp.full((16,), c, jnp.int32)
                    col = plsc.load_gather(table_v, [idxvec, csplat])
                    plsc.store_scatter(rows_ref, [rowvec, csplat], col)
                return carry
            lax.fori_loop(0, _GROUPS, jbody, 0)

        def step(g, b, not_first, not_last):
            pltpu.make_async_copy(
                idx_hbm.at[pl.ds(base(g), _CHUNK)], idx_b[b], si[b]).wait()

            def next_idx_load():
                pltpu.async_copy(
                    idx_hbm.at[pl.ds(base(g + 1), _CHUNK)],
                    idx_b[1 - b], si[1 - b])
            if not_last is None:
                next_idx_load()
            else:
                pl.when(not_last)(next_idx_load)

            def wait_prev_out():
                pltpu.make_async_copy(
                    rows_b[b], out_hbm.at[pl.ds(base(g - 2), _CHUNK)],
                    so[b]).wait()
            if not_first is None:
                wait_prev_out()
            else:
                pl.when(not_first)(wait_prev_out)

            compute_chunk(idx_b[b], rows_b[b])

            pltpu.async_copy(rows_b[b], out_hbm.at[pl.ds(base(g), _CHUNK)],
                             so[b])

        def body(i, carry):
            g = 2 * i
            step(g, 0, i >= 1, None)
            step(g + 1, 1, i >= 1, i < _NHALF - 1)
            return carry

        lax.fori_loop(0, _NHALF, body, 0)

        pltpu.make_async_copy(
            rows0, out_hbm.at[pl.ds(base(_NITER - 2), _CHUNK)], so0).wait()
        pltpu.make_async_copy(
            rows1, out_hbm.at[pl.ds(base(_NITER - 1), _CHUNK)], so1).wait()

    return gather_kernel


_gather = _make_kernel()


def kernel(edge_attr, table):
    return _gather(table, edge_attr)


# speedup vs baseline: 7.4139x; 1.2247x over previous
import functools

import jax
import jax.numpy as jnp
from jax import lax
from jax.experimental import pallas as pl
from jax.experimental.pallas import tpu as pltpu
from jax.experimental.pallas import tpu_sc as plsc

NUM_TYPES = 32
EMB_DIM = 16
N_EDGES = 3200000

_info = plsc.get_sparse_core_info()
_NC, _NS = _info.num_cores, _info.num_subcores
_NW = _NC * _NS
_PER_W = N_EDGES // _NW
_CHUNK = 2000
_NITER = _PER_W // _CHUNK
_NHALF = _NITER // 2
_GROUPS = _CHUNK // 16


def _make_kernel():
    mesh = plsc.VectorSubcoreMesh(core_axis_name="c", subcore_axis_name="s")

    @functools.partial(
        pl.kernel,
        mesh=mesh,
        compiler_params=pltpu.CompilerParams(
            use_tc_tiling_on_sc=False, needs_layout_passes=False),
        out_type=jax.ShapeDtypeStruct((N_EDGES * EMB_DIM,), jnp.float32),
        scratch_types=[
            pltpu.VMEM((NUM_TYPES, EMB_DIM), jnp.float32),
            pltpu.VMEM((_CHUNK,), jnp.int32),
            pltpu.VMEM((_CHUNK,), jnp.int32),
            pltpu.VMEM((_CHUNK * EMB_DIM,), jnp.float32),
            pltpu.VMEM((_CHUNK * EMB_DIM,), jnp.float32),
        ] + [pltpu.SemaphoreType.DMA] * 4,
    )
    def gather_kernel(table_hbm, idx_hbm, out_hbm,
                      table_v, idx0, idx1, rows0, rows1, si0, si1, so0, so1):
        wid = lax.axis_index("s") * _NC + lax.axis_index("c")
        w_base = wid * _PER_W
        idx_b, rows_b = (idx0, idx1), (rows0, rows1)
        si, so = (si0, si1), (so0, so1)

        def ibase(g):
            return w_base + g * _CHUNK

        def obase(g):
            return (w_base + g * _CHUNK) * EMB_DIM

        pltpu.sync_copy(table_hbm, table_v)
        pltpu.async_copy(idx_hbm.at[pl.ds(ibase(0), _CHUNK)], idx0, si0)

        lanes = lax.iota(jnp.int32, 16)
        ksplats = [jnp.full((16, 1), k, jnp.int32) for k in range(16)]
        dnums = lax.GatherDimensionNumbers(
            offset_dims=(), collapsed_slice_dims=(0,), start_index_map=(0,))

        def compute_chunk(idx_ref, rows_ref):
            def jbody(j, carry):
                idxvec = idx_ref[pl.ds(j * 16, 16)]
                rbase = j * 256
                for k in range(16):
                    bk = lax.gather(
                        idxvec, ksplats[k], dnums, slice_sizes=(1,),
                        mode=lax.GatherScatterMode.PROMISE_IN_BOUNDS)
                    row = plsc.load_gather(table_v, [bk, lanes])
                    rows_ref[pl.ds(rbase + k * 16, 16)] = row
                return carry
            lax.fori_loop(0, _GROUPS, jbody, 0)

        def step(g, b, not_first, not_last):
            pltpu.make_async_copy(
                idx_hbm.at[pl.ds(ibase(g), _CHUNK)], idx_b[b], si[b]).wait()

            def next_idx_load():
                pltpu.async_copy(
                    idx_hbm.at[pl.ds(ibase(g + 1), _CHUNK)],
                    idx_b[1 - b], si[1 - b])
            if not_last is None:
                next_idx_load()
            else:
                pl.when(not_last)(next_idx_load)

            def wait_prev_out():
                pltpu.make_async_copy(
                    rows_b[b],
                    out_hbm.at[pl.ds(obase(g - 2), _CHUNK * EMB_DIM)],
                    so[b]).wait()
            if not_first is None:
                wait_prev_out()
            else:
                pl.when(not_first)(wait_prev_out)

            compute_chunk(idx_b[b], rows_b[b])

            pltpu.async_copy(
                rows_b[b], out_hbm.at[pl.ds(obase(g), _CHUNK * EMB_DIM)],
                so[b])

        def body(i, carry):
            g = 2 * i
            step(g, 0, i >= 1, None)
            step(g + 1, 1, i >= 1, i < _NHALF - 1)
            return carry

        lax.fori_loop(0, _NHALF, body, 0)

        pltpu.make_async_copy(
            rows0, out_hbm.at[pl.ds(obase(_NITER - 2), _CHUNK * EMB_DIM)],
            so0).wait()
        pltpu.make_async_copy(
            rows1, out_hbm.at[pl.ds(obase(_NITER - 1), _CHUNK * EMB_DIM)],
            so1).wait()

    return gather_kernel


_gather = _make_kernel()


def kernel(edge_attr, table):
    return _gather(table, edge_attr).reshape(N_EDGES, EMB_DIM)
